# baseline (speedup 1.0000x reference)
import functools

import jax
import jax.numpy as jnp
from jax import lax
from jax.experimental import pallas as pl
from jax.experimental.pallas import tpu as pltpu
from jax.experimental.pallas import tpu_sc as plsc

N = 128 * 128
NZ = 1024
NCHUNK = 16
CZ = NZ // NCHUNK
NC, NS = 2, 16
B = 128
NNZ = 268435


def _matmul_body(x_ref, mz_ref, out_ref):
    r = lax.dot_general(
        x_ref[...], mz_ref[...], (((1,), (1,)), ((), ())),
        preferred_element_type=jnp.float32)
    for c in range(NCHUNK):
        out_ref[c] = r[:, c * CZ:(c + 1) * CZ]


def _matmul_chunked(squ, mat_z):
    BN = 2048
    return pl.pallas_call(
        _matmul_body,
        grid=(N // BN,),
        in_specs=[
            pl.BlockSpec((BN, NZ), lambda i: (i, 0)),
            pl.BlockSpec((NZ, NZ), lambda i: (0, 0)),
        ],
        out_specs=pl.BlockSpec((NCHUNK, BN, CZ), lambda i: (0, i, 0)),
        out_shape=jax.ShapeDtypeStruct((NCHUNK, N, CZ), jnp.float32),
    )(squ, mat_z)


def _sc_scatter(r1_flat, rows_p, cols_p, vals_p, nb):
    ew = nb * B
    zr = N // NS
    mesh = plsc.VectorSubcoreMesh(core_axis_name="c", subcore_axis_name="s")

    zb = 64
    NSLOT = 4

    @functools.partial(
        pl.kernel,
        out_type=jax.ShapeDtypeStruct((NCHUNK, N, CZ), jnp.float32),
        mesh=mesh,
        scratch_types=[
            pltpu.VMEM((ew,), jnp.int32),
            [pltpu.VMEM((B,), jnp.int32) for _ in range(NSLOT)],
            [pltpu.VMEM((B,), jnp.int32) for _ in range(NSLOT)],
            [pltpu.VMEM((B,), jnp.float32) for _ in range(NSLOT)],
            [pltpu.VMEM((B, CZ), jnp.float32) for _ in range(NSLOT)],
            pltpu.VMEM((zb, CZ), jnp.float32),
            [pltpu.SemaphoreType.DMA for _ in range(NSLOT)],
            [pltpu.SemaphoreType.DMA for _ in range(NSLOT)],
            [pltpu.SemaphoreType.DMA for _ in range(NSLOT)],
            pltpu.VMEM_SHARED((N, CZ), jnp.float32),
        ],
        compiler_params=pltpu.CompilerParams(use_tc_tiling_on_sc=False),
    )
    def k(r1_hbm, rows_hbm, cols_hbm, vals_hbm, out_hbm,
          rows_v, idx, cb, vb, buf, z_v, sg, sa, ss, acc):
        cid = lax.axis_index("c")
        sid = lax.axis_index("s")

        base = sid * ew
        pltpu.sync_copy(rows_hbm.at[pl.ds(base, ew)], rows_v)

        zero16 = jnp.zeros((16,), jnp.float32)

        def zfill(i, _):
            for j in range(CZ // 16):
                z_v[i, pl.ds(j * 16, 16)] = zero16
            return 0

        lax.fori_loop(0, zb, zfill, 0)

        def chunk_body(cc, _):
            c = cid * (NCHUNK // NC) + cc
            for h in range(zr // zb):
                pltpu.sync_copy(z_v, acc.at[pl.ds(sid * zr + h * zb, zb)])
            plsc.subcore_barrier()

            coff = c * N

            def fire(b, s):
                bb = b * B
                for g in range(B // 16):
                    sl = pl.ds(g * 16, 16)
                    idx[s][sl] = rows_v[pl.ds(bb + g * 16, 16)] + coff
                pltpu.async_copy(cols_hbm.at[pl.ds(base + bb, B)],
                                 cb[s], sa[s])
                pltpu.async_copy(vals_hbm.at[pl.ds(base + bb, B)],
                                 vb[s], sa[s])
                pltpu.async_copy(r1_hbm.at[idx[s]], buf[s], sg[s])

            def wait_in(s):
                pltpu.make_async_copy(r1_hbm.at[idx[s]], buf[s], sg[s]).wait()
                pltpu.make_async_copy(cols_hbm.at[pl.ds(0, B)],
                                      cb[s], sa[s]).wait()
                pltpu.make_async_copy(vals_hbm.at[pl.ds(0, B)],
                                      vb[s], sa[s]).wait()

            def wait_scat(s):
                pltpu.make_async_copy(buf[s], acc.at[cb[s]], ss[s]).wait()

            def mult(s):
                def ent_body(g, _):
                    vg = vb[s][pl.ds(g * 16, 16)]
                    for i16 in range(16):
                        vv = vg[i16]
                        i = g * 16 + i16
                        for j in range(CZ // 16):
                            sl = pl.ds(j * 16, 16)
                            buf[s][i, sl] = buf[s][i, sl] * vv
                    return 0

                lax.fori_loop(0, B // 16, ent_body, 0)

            fire(0, 0)
            fire(1, 1)

            def blk4(q, _):
                for s in range(NSLOT):
                    b = q * NSLOT + s
                    s2 = (s + 2) % NSLOT

                    @pl.when(b + 2 < nb)
                    def _():
                        fire(b + 2, s2)

                    wait_in(s)
                    mult(s)
                    pltpu.sync_copy(buf[s], acc.at[cb[s]], add=True)
                return 0

            lax.fori_loop(0, nb // NSLOT, blk4, 0)
            plsc.subcore_barrier()
            pltpu.sync_copy(acc.at[pl.ds(sid * zr, zr)],
                            out_hbm.at[c, pl.ds(sid * zr, zr)])
            plsc.subcore_barrier()
            return 0

        lax.fori_loop(0, NCHUNK // NC, chunk_body, 0)

    return k(r1_flat, rows_p, cols_p, vals_p)


def kernel(image, psf_rows, psf_cols, psf_vals, mat_z):
    squ = image.reshape(N, NZ)
    r1c = _matmul_chunked(squ, mat_z)
    r1_flat = r1c.reshape(NCHUNK * N, CZ)

    nb = -(-NNZ // (NS * B))
    nnz_pad = NS * nb * B
    pad = nnz_pad - NNZ
    rows_p = jnp.pad(psf_rows, (0, pad))
    cols_p = jnp.pad(psf_cols, (0, pad))
    vals_p = jnp.pad(psf_vals, (0, pad))

    outc = _sc_scatter(r1_flat, rows_p, cols_p, vals_p, nb)
    return outc.transpose(1, 0, 2).reshape(128, 128, NZ)


# speedup vs baseline: 1.8627x; 1.8627x over previous
import functools

import jax
import jax.numpy as jnp
from jax import lax
from jax.experimental import pallas as pl
from jax.experimental.pallas import tpu as pltpu
from jax.experimental.pallas import tpu_sc as plsc

N = 128 * 128
NZ = 1024
NCHUNK = 16
CZ = NZ // NCHUNK
NC, NS = 2, 16
B = 128
NNZ = 268435


def _matmul_body(x_ref, mz_ref, out_ref):
    r = lax.dot_general(
        x_ref[...], mz_ref[...], (((1,), (1,)), ((), ())),
        preferred_element_type=jnp.float32)
    for c in range(NCHUNK):
        out_ref[c] = r[:, c * CZ:(c + 1) * CZ]


def _matmul_chunked(squ, mat_z):
    BN = 2048
    return pl.pallas_call(
        _matmul_body,
        grid=(N // BN,),
        in_specs=[
            pl.BlockSpec((BN, NZ), lambda i: (i, 0)),
            pl.BlockSpec((NZ, NZ), lambda i: (0, 0)),
        ],
        out_specs=pl.BlockSpec((NCHUNK, BN, CZ), lambda i: (0, i, 0)),
        out_shape=jax.ShapeDtypeStruct((NCHUNK, N, CZ), jnp.float32),
    )(squ, mat_z)


def _sc_scatter(r1_flat, rows_p, cols_p, vals_p, nb):
    ew = nb * B
    zr = N // NS
    mesh = plsc.VectorSubcoreMesh(core_axis_name="c", subcore_axis_name="s")

    zb = 64
    NSLOT = 2

    @functools.partial(
        pl.kernel,
        out_type=jax.ShapeDtypeStruct((N, NCHUNK, CZ), jnp.float32),
        mesh=mesh,
        scratch_types=[
            pltpu.VMEM((ew,), jnp.int32),
            pltpu.VMEM((ew,), jnp.float32),
            [pltpu.VMEM((B,), jnp.int32) for _ in range(NSLOT)],
            [pltpu.VMEM((B,), jnp.int32) for _ in range(NSLOT)],
            [pltpu.VMEM((B, CZ), jnp.float32) for _ in range(NSLOT)],
            pltpu.VMEM((zb, CZ), jnp.float32),
            [pltpu.SemaphoreType.DMA for _ in range(NSLOT)],
            [pltpu.SemaphoreType.DMA for _ in range(NSLOT)],
            pltpu.VMEM_SHARED((N, CZ), jnp.float32),
        ],
        compiler_params=pltpu.CompilerParams(use_tc_tiling_on_sc=False),
    )
    def k(r1_hbm, rows_hbm, cols_hbm, vals_hbm, out_hbm,
          rows_v, vals_v, idx, cb, buf, z_v, sg, sa, acc):
        cid = lax.axis_index("c")
        sid = lax.axis_index("s")

        base = sid * ew
        pltpu.sync_copy(rows_hbm.at[pl.ds(base, ew)], rows_v)
        pltpu.sync_copy(vals_hbm.at[pl.ds(base, ew)], vals_v)

        zero16 = jnp.zeros((16,), jnp.float32)

        def zfill(i, _):
            for j in range(CZ // 16):
                z_v[i, pl.ds(j * 16, 16)] = zero16
            return 0

        lax.fori_loop(0, zb, zfill, 0)

        def chunk_body(cc, _):
            c = cid * (NCHUNK // NC) + cc
            for h in range(zr // zb):
                pltpu.sync_copy(z_v, acc.at[pl.ds(sid * zr + h * zb, zb)])
            plsc.subcore_barrier()

            coff = c * N

            def fire(b, s):
                bb = b * B
                for g in range(B // 16):
                    sl = pl.ds(g * 16, 16)
                    idx[s][sl] = rows_v[pl.ds(bb + g * 16, 16)] + coff
                pltpu.async_copy(cols_hbm.at[pl.ds(base + bb, B)],
                                 cb[s], sa[s])
                pltpu.async_copy(r1_hbm.at[idx[s]], buf[s], sg[s])

            def wait_in(s):
                pltpu.make_async_copy(r1_hbm.at[idx[s]], buf[s], sg[s]).wait()
                pltpu.make_async_copy(cols_hbm.at[pl.ds(0, B)],
                                      cb[s], sa[s]).wait()

            def process(b, s):
                bb = b * B

                def ent_body(g, _):
                    vg = vals_v[pl.ds(bb + g * 16, 16)]
                    for i16 in range(16):
                        vv = vg[i16]
                        i = g * 16 + i16
                        for j in range(CZ // 16):
                            sl = pl.ds(j * 16, 16)
                            buf[s][i, sl] = buf[s][i, sl] * vv
                    return 0

                lax.fori_loop(0, B // 16, ent_body, 0)
                pltpu.sync_copy(buf[s], acc.at[cb[s]], add=True)

            fire(0, 0)

            def blk2(b2, _):
                b = b2 * 2
                fire(b + 1, 1)
                wait_in(0)
                process(b, 0)

                @pl.when(b2 + 1 < nb // 2)
                def _():
                    fire(b + 2, 0)

                wait_in(1)
                process(b + 1, 1)
                return 0

            lax.fori_loop(0, nb // 2, blk2, 0)
            plsc.subcore_barrier()
            pltpu.sync_copy(acc.at[pl.ds(sid * zr, zr)],
                            out_hbm.at[pl.ds(sid * zr, zr), c])
            plsc.subcore_barrier()
            return 0

        lax.fori_loop(0, NCHUNK // NC, chunk_body, 0)

    return k(r1_flat, rows_p, cols_p, vals_p)


def kernel(image, psf_rows, psf_cols, psf_vals, mat_z):
    squ = image.reshape(N, NZ)
    r1c = _matmul_chunked(squ, mat_z)
    r1_flat = r1c.reshape(NCHUNK * N, CZ)

    nb = -(-NNZ // (NS * B))
    nnz_pad = NS * nb * B
    pad = nnz_pad - NNZ
    rows_p = jnp.pad(psf_rows, (0, pad))
    cols_p = jnp.pad(psf_cols, (0, pad))
    vals_p = jnp.pad(psf_vals, (0, pad))

    outc = _sc_scatter(r1_flat, rows_p, cols_p, vals_p, nb)
    return outc.reshape(128, 128, NZ)


# speedup vs baseline: 2.0690x; 1.1107x over previous
import functools

import jax
import jax.numpy as jnp
from jax import lax
from jax.experimental import pallas as pl
from jax.experimental.pallas import tpu as pltpu
from jax.experimental.pallas import tpu_sc as plsc

N = 128 * 128
NZ = 1024
NCHUNK = 16
CZ = NZ // NCHUNK
NC, NS = 2, 16
B = 128
NNZ = 268435


def _matmul_body(x_ref, mz_ref, out_ref):
    r = lax.dot_general(
        x_ref[...], mz_ref[...], (((1,), (1,)), ((), ())),
        preferred_element_type=jnp.float32)
    for c in range(NCHUNK):
        out_ref[c] = r[:, c * CZ:(c + 1) * CZ]


def _matmul_chunked(squ, mat_z):
    BN = 2048
    return pl.pallas_call(
        _matmul_body,
        grid=(N // BN,),
        in_specs=[
            pl.BlockSpec((BN, NZ), lambda i: (i, 0)),
            pl.BlockSpec((NZ, NZ), lambda i: (0, 0)),
        ],
        out_specs=pl.BlockSpec((NCHUNK, BN, CZ), lambda i: (0, i, 0)),
        out_shape=jax.ShapeDtypeStruct((NCHUNK, N, CZ), jnp.float32),
    )(squ, mat_z)


def _sc_scatter(r1_flat, rows_p, cols_p, vals_p, nb):
    ew = nb * B
    zr = N // NS
    mesh = plsc.VectorSubcoreMesh(core_axis_name="c", subcore_axis_name="s")

    zb = 64
    NSLOT = 2

    @functools.partial(
        pl.kernel,
        out_type=jax.ShapeDtypeStruct((N, NCHUNK, CZ), jnp.float32),
        mesh=mesh,
        scratch_types=[
            pltpu.VMEM((ew,), jnp.int32),
            pltpu.VMEM((ew,), jnp.float32),
            [pltpu.VMEM((B,), jnp.int32) for _ in range(NSLOT)],
            [pltpu.VMEM((B,), jnp.int32) for _ in range(NSLOT)],
            [pltpu.VMEM((B, CZ), jnp.float32) for _ in range(NSLOT)],
            pltpu.VMEM((zb, CZ), jnp.float32),
            [pltpu.SemaphoreType.DMA for _ in range(NSLOT)],
            [pltpu.SemaphoreType.DMA for _ in range(NSLOT)],
            pltpu.VMEM_SHARED((N, CZ), jnp.float32),
        ],
        compiler_params=pltpu.CompilerParams(use_tc_tiling_on_sc=False),
    )
    def k(r1_hbm, rows_hbm, cols_hbm, vals_hbm, out_hbm,
          rows_v, vals_v, idx, cb, buf, z_v, sg, sa, acc):
        cid = lax.axis_index("c")
        sid = lax.axis_index("s")

        base = sid * ew
        pltpu.sync_copy(rows_hbm.at[pl.ds(base, ew)], rows_v)
        pltpu.sync_copy(vals_hbm.at[pl.ds(base, ew)], vals_v)

        zero16 = jnp.zeros((16,), jnp.float32)

        def zfill(i, _):
            for j in range(CZ // 16):
                z_v[i, pl.ds(j * 16, 16)] = zero16
            return 0

        lax.fori_loop(0, zb, zfill, 0)

        def chunk_body(cc, _):
            c = cid * (NCHUNK // NC) + cc
            for h in range(zr // zb):
                pltpu.sync_copy(z_v, acc.at[pl.ds(sid * zr + h * zb, zb)])
            plsc.subcore_barrier()

            coff = c * N

            def fire(b, s):
                bb = b * B
                for g in range(B // 16):
                    sl = pl.ds(g * 16, 16)
                    idx[s][sl] = rows_v[pl.ds(bb + g * 16, 16)] + coff
                pltpu.async_copy(cols_hbm.at[pl.ds(base + bb, B)],
                                 cb[s], sa[s])
                pltpu.async_copy(r1_hbm.at[idx[s]], buf[s], sg[s])

            def wait_in(s):
                pltpu.make_async_copy(r1_hbm.at[idx[s]], buf[s], sg[s]).wait()
                pltpu.make_async_copy(cols_hbm.at[pl.ds(0, B)],
                                      cb[s], sa[s]).wait()

            def process(b, s):
                bb = b * B

                @plsc.parallel_loop(0, B // 16, unroll=2)
                def ent_body(g):
                    vg = vals_v[pl.ds(bb + g * 16, 16)]
                    for i16 in range(16):
                        vv = vg[i16]
                        i = g * 16 + i16
                        for j in range(CZ // 16):
                            sl = pl.ds(j * 16, 16)
                            buf[s][i, sl] = buf[s][i, sl] * vv

                pltpu.sync_copy(buf[s], acc.at[cb[s]], add=True)

            fire(0, 0)

            def blk2(b2, _):
                b = b2 * 2
                fire(b + 1, 1)
                wait_in(0)
                process(b, 0)

                @pl.when(b2 + 1 < nb // 2)
                def _():
                    fire(b + 2, 0)

                wait_in(1)
                process(b + 1, 1)
                return 0

            lax.fori_loop(0, nb // 2, blk2, 0)
            plsc.subcore_barrier()
            pltpu.sync_copy(acc.at[pl.ds(sid * zr, zr)],
                            out_hbm.at[pl.ds(sid * zr, zr), c])
            plsc.subcore_barrier()
            return 0

        lax.fori_loop(0, NCHUNK // NC, chunk_body, 0)

    return k(r1_flat, rows_p, cols_p, vals_p)


def kernel(image, psf_rows, psf_cols, psf_vals, mat_z):
    squ = image.reshape(N, NZ)
    r1c = _matmul_chunked(squ, mat_z)
    r1_flat = r1c.reshape(NCHUNK * N, CZ)

    nb = -(-NNZ // (NS * B))
    nnz_pad = NS * nb * B
    pad = nnz_pad - NNZ
    rows_p = jnp.pad(psf_rows, (0, pad))
    cols_p = jnp.pad(psf_cols, (0, pad))
    vals_p = jnp.pad(psf_vals, (0, pad))

    outc = _sc_scatter(r1_flat, rows_p, cols_p, vals_p, nb)
    return outc.reshape(128, 128, NZ)
